# baseline (speedup 1.0000x reference)
import functools

import jax
import jax.numpy as jnp
from jax import lax
from jax.experimental import pallas as pl
from jax.experimental.pallas import tpu as pltpu

NODE_HIDDEN_ = 128
E_ = 4


def _tc_body(d_enc_ref, hv_ref, W_ref, b_ref, lp_ref, chosen_ref):
    n_dests = hv_ref.shape[1] - 1
    hvb = hv_ref[0]
    dests = hvb[:n_dests, :]
    src = hvb[n_dests:, :]
    W = W_ref[...]
    Wd = W[:, :NODE_HIDDEN_]
    Ws = W[:, NODE_HIDDEN_:]
    sd = lax.dot_general(dests, Wd, (((1,), (1,)), ((), ())),
                         preferred_element_type=jnp.float32)
    ss = lax.dot_general(src, Ws, (((1,), (1,)), ((), ())),
                         preferred_element_type=jnp.float32)
    scores = sd + ss + b_ref[...]
    m = jnp.max(scores)
    ex = jnp.exp(scores - m)
    lse = m + jnp.log(jnp.sum(ex))
    lp = scores - lse
    lp_ref[0] = lp
    de = d_enc_ref[pl.program_id(0)]
    flat_idx = (lax.broadcasted_iota(jnp.int32, (n_dests, E_), 0) * E_
                + lax.broadcasted_iota(jnp.int32, (n_dests, E_), 1))
    chosen_ref[0, 0, 0] = jnp.sum(jnp.where(flat_idx == de, lp, 0.0))


def kernel(hv, d_enc, W, b):
    B, N, D = hv.shape
    n_dests = N - 1
    lp, chosen = pl.pallas_call(
        _tc_body,
        grid=(B,),
        in_specs=[
            pl.BlockSpec(memory_space=pltpu.SMEM),
            pl.BlockSpec((1, N, D), lambda i: (i, 0, 0)),
            pl.BlockSpec((E_, 2 * D), lambda i: (0, 0)),
            pl.BlockSpec((1, E_), lambda i: (0, 0)),
        ],
        out_specs=[
            pl.BlockSpec((1, n_dests, E_), lambda i: (i, 0, 0)),
            pl.BlockSpec((1, 1, 1), lambda i: (i, 0, 0),
                         memory_space=pltpu.SMEM),
        ],
        out_shape=[
            jax.ShapeDtypeStruct((B, n_dests, E_), jnp.float32),
            jax.ShapeDtypeStruct((B, 1, 1), jnp.float32),
        ],
    )(d_enc, hv, W, b[None, :])
    return lp.reshape(B, n_dests * E_), chosen.reshape(B, 1)


# speedup vs baseline: 1.1117x; 1.1117x over previous
import functools

import jax
import jax.numpy as jnp
from jax import lax
from jax.experimental import pallas as pl
from jax.experimental.pallas import tpu as pltpu

NODE_HIDDEN_ = 128
E_ = 4


def _tc_body(d_enc_ref, hv_ref, W_ref, b_ref, lp_ref, chosen_ref):
    n_dests = hv_ref.shape[1] - 1
    hvb = hv_ref[0]
    dests = hvb[:n_dests, :]
    src = hvb[n_dests:, :]
    W = W_ref[...]
    Wd = W[:, :NODE_HIDDEN_]
    Ws = W[:, NODE_HIDDEN_:]
    sd = lax.dot_general(Wd, dests, (((1,), (1,)), ((), ())),
                         preferred_element_type=jnp.float32)
    ss = lax.dot_general(Ws, src, (((1,), (1,)), ((), ())),
                         preferred_element_type=jnp.float32)
    scores = sd + ss + b_ref[...]
    m = jnp.max(scores)
    ex = jnp.exp(scores - m)
    lse = m + jnp.log(jnp.sum(ex))
    lp = scores - lse
    lp_ref[0] = lp.T
    de = d_enc_ref[pl.program_id(0)]
    flat_idx = (lax.broadcasted_iota(jnp.int32, (E_, n_dests), 1) * E_
                + lax.broadcasted_iota(jnp.int32, (E_, n_dests), 0))
    chosen_ref[0, 0, 0] = jnp.sum(jnp.where(flat_idx == de, lp, 0.0))


def kernel(hv, d_enc, W, b):
    B, N, D = hv.shape
    n_dests = N - 1
    lp, chosen = pl.pallas_call(
        _tc_body,
        grid=(B,),
        in_specs=[
            pl.BlockSpec(memory_space=pltpu.SMEM),
            pl.BlockSpec((1, N, D), lambda i: (i, 0, 0)),
            pl.BlockSpec((E_, 2 * D), lambda i: (0, 0)),
            pl.BlockSpec((E_, 1), lambda i: (0, 0)),
        ],
        out_specs=[
            pl.BlockSpec((1, n_dests, E_), lambda i: (i, 0, 0)),
            pl.BlockSpec((1, 1, 1), lambda i: (i, 0, 0),
                         memory_space=pltpu.SMEM),
        ],
        out_shape=[
            jax.ShapeDtypeStruct((B, n_dests, E_), jnp.float32),
            jax.ShapeDtypeStruct((B, 1, 1), jnp.float32),
        ],
    )(d_enc, hv, W, b[:, None])
    return lp.reshape(B, n_dests * E_), chosen.reshape(B, 1)


# speedup vs baseline: 3.0884x; 2.7780x over previous
import functools

import jax
import jax.numpy as jnp
from jax import lax
from jax.experimental import pallas as pl
from jax.experimental.pallas import tpu as pltpu

NODE_HIDDEN_ = 128
E_ = 4


def _tc_body(d_enc_ref, hv_ref, W_ref, b_ref, lp_ref, chosen_ref):
    n_dests = hv_ref.shape[1] - 1
    hvb = hv_ref[0]
    dests = hvb[:n_dests, :]
    src = hvb[n_dests:, :]
    W = W_ref[...]
    Wd = W[:, :NODE_HIDDEN_]
    Ws = W[:, NODE_HIDDEN_:]
    sd = lax.dot_general(Wd, dests, (((1,), (1,)), ((), ())),
                         preferred_element_type=jnp.float32)
    ss = lax.dot_general(Ws, src, (((1,), (1,)), ((), ())),
                         preferred_element_type=jnp.float32)
    scores = sd + ss + b_ref[...]
    m = jnp.max(scores)
    ex = jnp.exp(scores - m)
    lse = m + jnp.log(jnp.sum(ex))
    lp = scores - lse
    lp_ref[0] = lp
    de = d_enc_ref[pl.program_id(0)]
    flat_idx = (lax.broadcasted_iota(jnp.int32, (E_, n_dests), 1) * E_
                + lax.broadcasted_iota(jnp.int32, (E_, n_dests), 0))
    chosen_ref[0, 0, 0] = jnp.sum(jnp.where(flat_idx == de, lp, 0.0))


def kernel(hv, d_enc, W, b):
    B, N, D = hv.shape
    n_dests = N - 1
    lp, chosen = pl.pallas_call(
        _tc_body,
        grid=(B,),
        in_specs=[
            pl.BlockSpec(memory_space=pltpu.SMEM),
            pl.BlockSpec((1, N, D), lambda i: (i, 0, 0)),
            pl.BlockSpec((E_, 2 * D), lambda i: (0, 0)),
            pl.BlockSpec((E_, 1), lambda i: (0, 0)),
        ],
        out_specs=[
            pl.BlockSpec((1, E_, n_dests), lambda i: (i, 0, 0)),
            pl.BlockSpec((1, 1, 1), lambda i: (i, 0, 0),
                         memory_space=pltpu.SMEM),
        ],
        out_shape=[
            jax.ShapeDtypeStruct((B, E_, n_dests), jnp.float32),
            jax.ShapeDtypeStruct((B, 1, 1), jnp.float32),
        ],
    )(d_enc, hv, W, b[:, None])
    lp_flat = lp.transpose(0, 2, 1).reshape(B, n_dests * E_)
    return lp_flat, chosen.reshape(B, 1)
